# Initial kernel scaffold; baseline (speedup 1.0000x reference)
import jax
import jax.numpy as jnp
from jax.experimental import pallas as pl


def kernel(x, edge_index, weight, Wl, Wr, b, Wl9, Wr9, b9):
    raise NotImplementedError("write your pallas kernel here")


# speedup vs baseline: 4.2204x; 4.2204x over previous
import functools

import jax
import jax.numpy as jnp
from jax import lax
from jax.experimental import pallas as pl
from jax.experimental.pallas import tpu as pltpu
from jax.experimental.pallas import tpu_sc as plsc

_N = 10000
_E = 320000
_D = 128
_C = 40
_NC, _NS = 2, 16
_NW = _NC * _NS
_K = 128
_CH = 79
_EPW = _CH * _K
_NPAD = 10112
_RPS = _NPAD // _NS
_DUMMY = _N

_mesh = plsc.VectorSubcoreMesh(
    core_axis_name="c", subcore_axis_name="s", num_cores=_NC, num_subcores=_NS
)


def _agg_body(h_hbm, src_hbm, dst_hbm, zeros_hbm, acc_hbm,
              src_v, dst_v, buf, acc_sh, gsem):
    c = lax.axis_index("c")
    s = lax.axis_index("s")
    rows = pl.ds(s * _RPS, _RPS)
    pltpu.sync_copy(src_hbm.at[c, s], src_v)
    pltpu.sync_copy(dst_hbm.at[c, s], dst_v)
    pltpu.sync_copy(zeros_hbm.at[rows], acc_sh.at[rows])
    plsc.subcore_barrier()

    def body(j, carry):
        pltpu.async_copy(h_hbm.at[src_v.at[j]], buf, gsem).wait()
        pltpu.sync_copy(buf, acc_sh.at[dst_v.at[j]], add=True)
        return carry

    lax.fori_loop(0, _CH, body, 0)
    plsc.subcore_barrier()
    pltpu.sync_copy(acc_sh.at[rows], acc_hbm.at[c, rows])


_agg_call = functools.partial(
    pl.kernel,
    _agg_body,
    out_type=jax.ShapeDtypeStruct((_NC, _NPAD, _D), jnp.float32),
    mesh=_mesh,
    scratch_types=[
        pltpu.VMEM((_CH, _K), jnp.int32),
        pltpu.VMEM((_CH, _K), jnp.int32),
        pltpu.VMEM((_K, _D), jnp.float32),
        pltpu.VMEM_SHARED((_NPAD, _D), jnp.float32),
        pltpu.SemaphoreType.DMA,
    ],
)()


def _deg_body(dst_hbm, ones_hbm, zeros_hbm, deg_hbm, dst_v, ones_v, deg_sh):
    c = lax.axis_index("c")
    s = lax.axis_index("s")
    rows = pl.ds(s * _RPS, _RPS)
    pltpu.sync_copy(dst_hbm.at[c, s], dst_v)
    pltpu.sync_copy(ones_hbm, ones_v)
    pltpu.sync_copy(zeros_hbm.at[rows], deg_sh.at[rows])
    plsc.subcore_barrier()

    def body(j, carry):
        pltpu.sync_copy(ones_v, deg_sh.at[dst_v.at[j]], add=True)
        return carry

    lax.fori_loop(0, _CH, body, 0)
    plsc.subcore_barrier()
    pltpu.sync_copy(deg_sh.at[rows], deg_hbm.at[c, rows])


_deg_call = functools.partial(
    pl.kernel,
    _deg_body,
    out_type=jax.ShapeDtypeStruct((_NC, _NPAD, _D), jnp.float32),
    mesh=_mesh,
    scratch_types=[
        pltpu.VMEM((_CH, _K), jnp.int32),
        pltpu.VMEM((_K, _D), jnp.float32),
        pltpu.VMEM_SHARED((_NPAD, _D), jnp.float32),
    ],
)()


_BM = 1000


def _layer_body(acc_ref, deg_ref, h_ref, wl_ref, wr_ref, b_ref, out_ref, *, act):
    a = acc_ref[0] + acc_ref[1]
    dg = deg_ref[0, :, 0:1] + deg_ref[1, :, 0:1]
    mean = a * (1.0 / jnp.maximum(dg, 1.0))
    z = (jnp.dot(mean, wl_ref[...], preferred_element_type=jnp.float32)
         + jnp.dot(h_ref[...], wr_ref[...], preferred_element_type=jnp.float32)
         + b_ref[...])
    if act:
        z = jnp.where(z > 0, z, jnp.exp(z) - 1.0)
    out_ref[...] = z


def _layer_call(acc, deg, h, wl, wr, bias, act):
    return pl.pallas_call(
        functools.partial(_layer_body, act=act),
        grid=(_N // _BM,),
        in_specs=[
            pl.BlockSpec((_NC, _BM, _D), lambda i: (0, i, 0)),
            pl.BlockSpec((_NC, _BM, _D), lambda i: (0, i, 0)),
            pl.BlockSpec((_BM, _D), lambda i: (i, 0)),
            pl.BlockSpec((_D, _D), lambda i: (0, 0)),
            pl.BlockSpec((_D, _D), lambda i: (0, 0)),
            pl.BlockSpec((1, _D), lambda i: (0, 0)),
        ],
        out_specs=pl.BlockSpec((_BM, _D), lambda i: (i, 0)),
        out_shape=jax.ShapeDtypeStruct((_N, _D), jnp.float32),
    )(acc, deg, h, wl, wr, bias)


def kernel(x, edge_index, weight, Wl, Wr, b, Wl9, Wr9, b9):
    del weight
    src = edge_index[0].astype(jnp.int32)
    dst = edge_index[1].astype(jnp.int32)
    pad = _NW * _EPW - _E
    src_p = jnp.concatenate([src, jnp.zeros((pad,), jnp.int32)])
    src_p = src_p.reshape(_NC, _NS, _CH, _K)
    dst_p = jnp.concatenate([dst, jnp.full((pad,), _DUMMY, jnp.int32)])
    dst_p = dst_p.reshape(_NC, _NS, _CH, _K)

    zeros128 = jnp.zeros((_NPAD, _D), jnp.float32)
    ones128 = jnp.ones((_K, _D), jnp.float32)

    deg = _deg_call(dst_p, ones128, zeros128)

    h = x.astype(jnp.float32)
    for i in range(8):
        acc = _agg_call(h, src_p, dst_p, zeros128)
        h = _layer_call(acc, deg, h, Wl[i], Wr[i], b[i][None, :], act=True)

    acc = _agg_call(h, src_p, dst_p, zeros128)
    wl9 = jnp.pad(Wl9, ((0, 0), (0, _D - _C)))
    wr9 = jnp.pad(Wr9, ((0, 0), (0, _D - _C)))
    b9p = jnp.pad(b9, (0, _D - _C))[None, :]
    out = _layer_call(acc, deg, h, wl9, wr9, b9p, act=False)
    return out[:, :_C]
